# Initial kernel scaffold; baseline (speedup 1.0000x reference)
import jax
import jax.numpy as jnp
from jax.experimental import pallas as pl


def kernel(x, edge_index, W1, b1, W2, b2, W3, b3):
    raise NotImplementedError("write your pallas kernel here")


# speedup vs baseline: 9.5048x; 9.5048x over previous
import functools

import jax
import jax.numpy as jnp
from jax import lax
from jax.experimental import pallas as pl
from jax.experimental.pallas import tpu as pltpu
from jax.experimental.pallas import tpu_sc as plsc

N_NODES = 10000
N_EDGES = 320000
IN_FEATS = 128
N_HIDDEN = 128
N_CLASSES = 40

NC = 2
NS = 16
NW = NC * NS

K = 128
CH = -(-N_EDGES // (NW * K))
E_PAD = NW * CH * K
N_PAD = 10240
RPT = N_PAD // NS
BR = 512
GRID = N_PAD // BR

_f32 = jnp.float32


def _zero_fill(ref, n_rows):
    z = jnp.zeros((16,), _f32)

    def body(i, _):
        ref[i] = z
        return 0

    lax.fori_loop(0, n_rows, body, 0)


def _fill_ones(ref, n_rows):
    o = jnp.ones((16,), _f32)

    def body(i, _):
        ref[i] = o
        return 0

    lax.fori_loop(0, n_rows, body, 0)


def _make_deg_kernel():
    mesh = plsc.VectorSubcoreMesh(core_axis_name="c", subcore_axis_name="s")

    @functools.partial(
        pl.kernel,
        out_type=(
            jax.ShapeDtypeStruct((N_PAD, 16), _f32),
            jax.ShapeDtypeStruct((N_PAD, 16), _f32),
        ),
        mesh=mesh,
        scratch_types=[
            pltpu.VMEM((CH, K), jnp.int32),
            pltpu.VMEM((K, 16), _f32),
            pltpu.VMEM((16, 16), _f32),
            pltpu.VMEM_SHARED((N_PAD, 16), _f32),
        ],
    )
    def deg_kernel(dst_hbm, out0, out1, idx_v, ones_v, zeros_v, hist_sh):
        c = lax.axis_index("c")
        s = lax.axis_index("s")
        w = c * NS + s

        _fill_ones(ones_v, K)
        _zero_fill(zeros_v, 16)

        def zbody(i, _):
            pltpu.sync_copy(zeros_v, hist_sh.at[pl.ds(s * RPT + i * 16, 16)])
            return 0

        lax.fori_loop(0, RPT // 16, zbody, 0)
        plsc.subcore_barrier()

        pltpu.sync_copy(dst_hbm.at[w], idx_v)

        def body(ch, _):
            pltpu.sync_copy(ones_v, hist_sh.at[idx_v.at[ch]], add=True)
            return 0

        lax.fori_loop(0, CH, body, 0)
        plsc.subcore_barrier()

        @pl.when(c == 0)
        def _():
            pltpu.sync_copy(hist_sh.at[pl.ds(s * RPT, RPT)],
                            out0.at[pl.ds(s * RPT, RPT)])

        @pl.when(c == 1)
        def _():
            pltpu.sync_copy(hist_sh.at[pl.ds(s * RPT, RPT)],
                            out1.at[pl.ds(s * RPT, RPT)])

    return deg_kernel


def _make_prop_kernel(d_feat):
    mesh = plsc.VectorSubcoreMesh(core_axis_name="c", subcore_axis_name="s")

    @functools.partial(
        pl.kernel,
        out_type=(
            jax.ShapeDtypeStruct((N_PAD, d_feat), _f32),
            jax.ShapeDtypeStruct((N_PAD, d_feat), _f32),
        ),
        mesh=mesh,
        scratch_types=[
            pltpu.VMEM((CH, K), jnp.int32),
            pltpu.VMEM((CH, K), jnp.int32),
            pltpu.VMEM((K, d_feat), _f32),
            pltpu.VMEM((16, d_feat), _f32),
            pltpu.VMEM_SHARED((N_PAD, d_feat), _f32),
        ],
    )
    def prop_kernel(src_hbm, dst_hbm, hs_hbm, out0, out1,
                    src_v, dst_v, msg_v, zeros_v, acc_sh):
        c = lax.axis_index("c")
        s = lax.axis_index("s")
        w = c * NS + s

        ncol = d_feat // 16
        z16 = jnp.zeros((16,), _f32)

        def zfill(i, _):
            zeros_v[i // ncol, pl.ds((i % ncol) * 16, 16)] = z16
            return 0

        lax.fori_loop(0, 16 * ncol, zfill, 0)

        def zbody(i, _):
            pltpu.sync_copy(zeros_v, acc_sh.at[pl.ds(s * RPT + i * 16, 16)])
            return 0

        lax.fori_loop(0, RPT // 16, zbody, 0)
        plsc.subcore_barrier()

        pltpu.sync_copy(src_hbm.at[w], src_v)
        pltpu.sync_copy(dst_hbm.at[w], dst_v)

        def body(ch, _):
            pltpu.sync_copy(hs_hbm.at[src_v.at[ch]], msg_v)
            pltpu.sync_copy(msg_v, acc_sh.at[dst_v.at[ch]], add=True)
            return 0

        lax.fori_loop(0, CH, body, 0)
        plsc.subcore_barrier()

        @pl.when(c == 0)
        def _():
            pltpu.sync_copy(acc_sh.at[pl.ds(s * RPT, RPT)],
                            out0.at[pl.ds(s * RPT, RPT)])

        @pl.when(c == 1)
        def _():
            pltpu.sync_copy(acc_sh.at[pl.ds(s * RPT, RPT)],
                            out1.at[pl.ds(s * RPT, RPT)])

    return prop_kernel


def _dis(d0, d1):
    return lax.rsqrt(d0[:, 0:1] + d1[:, 0:1] + 1.0)


def _mm_first(x_ref, d0_ref, d1_ref, w_ref, o_ref):
    dis = _dis(d0_ref[...], d1_ref[...])
    h = jnp.dot(x_ref[...], w_ref[...], preferred_element_type=_f32)
    o_ref[...] = h * dis


def _mm_mid(a0_ref, a1_ref, hs_ref, d0_ref, d1_ref, b_ref, w_ref, o_ref):
    dis = _dis(d0_ref[...], d1_ref[...])
    a = (a0_ref[...] + a1_ref[...] + hs_ref[...]) * dis + b_ref[...]
    a = jnp.maximum(a, 0.0)
    h = jnp.dot(a, w_ref[...], preferred_element_type=_f32)
    o_ref[...] = h * dis


def _mm_final(a0_ref, a1_ref, hs_ref, d0_ref, d1_ref, b_ref, o_ref):
    dis = _dis(d0_ref[...], d1_ref[...])
    res = (a0_ref[...] + a1_ref[...] + hs_ref[...]) * dis + b_ref[...]
    o_ref[...] = res[:, :N_CLASSES]


def _row_spec(d):
    return pl.BlockSpec((BR, d), lambda i: (i, 0))


def _whole_spec(r, d):
    return pl.BlockSpec((r, d), lambda i: (0, 0))


def _call_mm_first(x_p, deg0, deg1, W):
    return pl.pallas_call(
        _mm_first,
        grid=(GRID,),
        in_specs=[_row_spec(IN_FEATS), _row_spec(16), _row_spec(16),
                  _whole_spec(IN_FEATS, N_HIDDEN)],
        out_specs=_row_spec(N_HIDDEN),
        out_shape=jax.ShapeDtypeStruct((N_PAD, N_HIDDEN), _f32),
    )(x_p, deg0, deg1, W)


def _call_mm_mid(a0, a1, hs, deg0, deg1, b, W, d_out):
    return pl.pallas_call(
        _mm_mid,
        grid=(GRID,),
        in_specs=[_row_spec(N_HIDDEN), _row_spec(N_HIDDEN),
                  _row_spec(N_HIDDEN), _row_spec(16), _row_spec(16),
                  _whole_spec(1, N_HIDDEN), _whole_spec(N_HIDDEN, d_out)],
        out_specs=_row_spec(d_out),
        out_shape=jax.ShapeDtypeStruct((N_PAD, d_out), _f32),
    )(a0, a1, hs, deg0, deg1, b, W)


def _call_mm_final(a0, a1, hs, deg0, deg1, b):
    return pl.pallas_call(
        _mm_final,
        grid=(GRID,),
        in_specs=[_row_spec(N_HIDDEN), _row_spec(N_HIDDEN),
                  _row_spec(N_HIDDEN), _row_spec(16), _row_spec(16),
                  _whole_spec(1, N_HIDDEN)],
        out_specs=_row_spec(N_CLASSES),
        out_shape=jax.ShapeDtypeStruct((N_NODES, N_CLASSES), _f32),
    )(a0, a1, hs, deg0, deg1, b)


@jax.jit
def kernel(x, edge_index, W1, b1, W2, b2, W3, b3):
    src = edge_index[0].astype(jnp.int32)
    dst = edge_index[1].astype(jnp.int32)
    pad = jnp.full((E_PAD - N_EDGES,), N_NODES, jnp.int32)
    src_p = jnp.concatenate([src, pad]).reshape(NW, CH, K)
    dst_p = jnp.concatenate([dst, pad]).reshape(NW, CH, K)

    x_p = jnp.pad(x, ((0, N_PAD - N_NODES), (0, 0)))
    W3p = jnp.pad(W3, ((0, 0), (0, N_HIDDEN - N_CLASSES)))
    b1r = b1.reshape(1, N_HIDDEN)
    b2r = b2.reshape(1, N_HIDDEN)
    b3r = jnp.pad(b3, (0, N_HIDDEN - N_CLASSES)).reshape(1, N_HIDDEN)

    deg0, deg1 = _make_deg_kernel()(dst_p)

    prop128 = _make_prop_kernel(N_HIDDEN)

    hs1 = _call_mm_first(x_p, deg0, deg1, W1)
    a10, a11 = prop128(src_p, dst_p, hs1)
    hs2 = _call_mm_mid(a10, a11, hs1, deg0, deg1, b1r, W2, N_HIDDEN)
    a20, a21 = prop128(src_p, dst_p, hs2)
    hs3 = _call_mm_mid(a20, a21, hs2, deg0, deg1, b2r, W3p, N_HIDDEN)
    a30, a31 = prop128(src_p, dst_p, hs3)
    return _call_mm_final(a30, a31, hs3, deg0, deg1, b3r)


# speedup vs baseline: 12.0489x; 1.2677x over previous
import functools

import jax
import jax.numpy as jnp
from jax import lax
from jax.experimental import pallas as pl
from jax.experimental.pallas import tpu as pltpu
from jax.experimental.pallas import tpu_sc as plsc

N_NODES = 10000
N_EDGES = 320000
IN_FEATS = 128
N_HIDDEN = 128
N_CLASSES = 40

NC = 2
NS = 16
NW = NC * NS

K = 128
CH = -(-N_EDGES // (NW * K))
E_PAD = NW * CH * K
N_PAD = 10240
RPT = N_PAD // NS
BR = 512
GRID = N_PAD // BR

_f32 = jnp.float32


def _zero_fill(ref, n_rows):
    z = jnp.zeros((16,), _f32)

    def body(i, _):
        ref[i] = z
        return 0

    lax.fori_loop(0, n_rows, body, 0)


def _fill_ones(ref, n_rows):
    o = jnp.ones((16,), _f32)

    def body(i, _):
        ref[i] = o
        return 0

    lax.fori_loop(0, n_rows, body, 0)


def _make_deg_kernel():
    mesh = plsc.VectorSubcoreMesh(core_axis_name="c", subcore_axis_name="s")

    @functools.partial(
        pl.kernel,
        out_type=(
            jax.ShapeDtypeStruct((N_PAD, 16), _f32),
            jax.ShapeDtypeStruct((N_PAD, 16), _f32),
        ),
        mesh=mesh,
        scratch_types=[
            pltpu.VMEM((CH, K), jnp.int32),
            pltpu.VMEM((K, 16), _f32),
            pltpu.VMEM((16, 16), _f32),
            pltpu.VMEM_SHARED((N_PAD, 16), _f32),
        ],
    )
    def deg_kernel(dst_hbm, out0, out1, idx_v, ones_v, zeros_v, hist_sh):
        c = lax.axis_index("c")
        s = lax.axis_index("s")
        w = c * NS + s

        _fill_ones(ones_v, K)
        _zero_fill(zeros_v, 16)

        def zbody(i, _):
            pltpu.sync_copy(zeros_v, hist_sh.at[pl.ds(s * RPT + i * 16, 16)])
            return 0

        lax.fori_loop(0, RPT // 16, zbody, 0)
        plsc.subcore_barrier()

        pltpu.sync_copy(dst_hbm.at[w], idx_v)

        def body(ch, _):
            pltpu.sync_copy(ones_v, hist_sh.at[idx_v.at[ch]], add=True)
            return 0

        lax.fori_loop(0, CH, body, 0)
        plsc.subcore_barrier()

        @pl.when(c == 0)
        def _():
            pltpu.sync_copy(hist_sh.at[pl.ds(s * RPT, RPT)],
                            out0.at[pl.ds(s * RPT, RPT)])

        @pl.when(c == 1)
        def _():
            pltpu.sync_copy(hist_sh.at[pl.ds(s * RPT, RPT)],
                            out1.at[pl.ds(s * RPT, RPT)])

    return deg_kernel


def _make_prop_kernel(d_feat):
    mesh = plsc.VectorSubcoreMesh(core_axis_name="c", subcore_axis_name="s")

    @functools.partial(
        pl.kernel,
        out_type=(
            jax.ShapeDtypeStruct((N_PAD, d_feat), _f32),
            jax.ShapeDtypeStruct((N_PAD, d_feat), _f32),
        ),
        mesh=mesh,
        scratch_types=[
            pltpu.VMEM((CH, K), jnp.int32),
            pltpu.VMEM((4, K), jnp.int32),
            pltpu.VMEM((2, K, d_feat), _f32),
            pltpu.VMEM_SHARED((N_PAD, d_feat), _f32),
            pltpu.SemaphoreType.DMA,
            pltpu.SemaphoreType.DMA,
            pltpu.SemaphoreType.DMA,
            pltpu.SemaphoreType.DMA,
            pltpu.SemaphoreType.DMA,
            pltpu.SemaphoreType.DMA,
        ],
    )
    def prop_kernel(src_hbm, dst_hbm, hs_hbm, out0, out1,
                    dst_v, ibuf, msg_v, acc_sh,
                    gsem0, gsem1, isem0, isem1, isem2, isem3):
        c = lax.axis_index("c")
        s = lax.axis_index("s")
        w = c * NS + s

        pltpu.sync_copy(dst_hbm.at[w], dst_v)

        ncol = d_feat // 16
        z16 = jnp.zeros((16,), _f32)

        def zfill(i, _):
            msg_v[0, i // ncol, pl.ds((i % ncol) * 16, 16)] = z16
            return 0

        lax.fori_loop(0, K * ncol, zfill, 0)

        def zbody(i, _):
            pltpu.sync_copy(msg_v.at[0], acc_sh.at[pl.ds(s * RPT + i * K, K)])
            return 0

        lax.fori_loop(0, RPT // K, zbody, 0)

        gsems = (gsem0, gsem1)
        isems = (isem0, isem1, isem2, isem3)

        def i_start(j, q):
            pltpu.async_copy(src_hbm.at[w * CH + j], ibuf.at[q], isems[q])

        def i_wait(j, q):
            pltpu.make_async_copy(src_hbm.at[w * CH + j], ibuf.at[q],
                                  isems[q]).wait()

        def g_start(j, q, b):
            pltpu.async_copy(hs_hbm.at[ibuf.at[q]], msg_v.at[b], gsems[b])

        def g_wait(j, q, b):
            pltpu.make_async_copy(hs_hbm.at[ibuf.at[q]], msg_v.at[b],
                                  gsems[b]).wait()

        def scat(j, b):
            pltpu.sync_copy(msg_v.at[b], acc_sh.at[dst_v.at[j]], add=True)

        for q in range(4):
            i_start(q, q)
        i_wait(0, 0)
        plsc.subcore_barrier()
        g_start(0, 0, 0)

        def body(it, _):
            j0 = it * 4
            for t in range(4):
                j = j0 + t
                sl_n = (t + 1) % 4
                mb = t % 2
                mb_n = (t + 1) % 2

                @pl.when(j + 1 < CH)
                def _(j=j, sl_n=sl_n, mb_n=mb_n):
                    i_wait(j + 1, sl_n)
                    g_start(j + 1, sl_n, mb_n)

                @pl.when(j < CH)
                def _(j=j, t=t, mb=mb):
                    g_wait(j, t, mb)
                    scat(j, mb)

                @pl.when(j + 4 < CH)
                def _(j=j, t=t):
                    i_start(j + 4, t)

            return 0

        lax.fori_loop(0, -(-CH // 4), body, 0)
        plsc.subcore_barrier()

        @pl.when(c == 0)
        def _():
            pltpu.sync_copy(acc_sh.at[pl.ds(s * RPT, RPT)],
                            out0.at[pl.ds(s * RPT, RPT)])

        @pl.when(c == 1)
        def _():
            pltpu.sync_copy(acc_sh.at[pl.ds(s * RPT, RPT)],
                            out1.at[pl.ds(s * RPT, RPT)])

    return prop_kernel


def _dis(d0, d1):
    return lax.rsqrt(d0[:, 0:1] + d1[:, 0:1] + 1.0)


def _mm_first(x_ref, d0_ref, d1_ref, w_ref, o_ref):
    dis = _dis(d0_ref[...], d1_ref[...])
    h = jnp.dot(x_ref[...], w_ref[...], preferred_element_type=_f32)
    o_ref[...] = h * dis


def _mm_mid(a0_ref, a1_ref, hs_ref, d0_ref, d1_ref, b_ref, w_ref, o_ref):
    dis = _dis(d0_ref[...], d1_ref[...])
    a = (a0_ref[...] + a1_ref[...] + hs_ref[...]) * dis + b_ref[...]
    a = jnp.maximum(a, 0.0)
    h = jnp.dot(a, w_ref[...], preferred_element_type=_f32)
    o_ref[...] = h * dis


def _mm_final(a0_ref, a1_ref, hs_ref, d0_ref, d1_ref, b_ref, o_ref):
    dis = _dis(d0_ref[...], d1_ref[...])
    res = (a0_ref[...] + a1_ref[...] + hs_ref[...]) * dis + b_ref[...]
    o_ref[...] = res[:, :N_CLASSES]


def _row_spec(d):
    return pl.BlockSpec((BR, d), lambda i: (i, 0))


def _whole_spec(r, d):
    return pl.BlockSpec((r, d), lambda i: (0, 0))


def _call_mm_first(x_p, deg0, deg1, W):
    return pl.pallas_call(
        _mm_first,
        grid=(GRID,),
        in_specs=[_row_spec(IN_FEATS), _row_spec(16), _row_spec(16),
                  _whole_spec(IN_FEATS, N_HIDDEN)],
        out_specs=_row_spec(N_HIDDEN),
        out_shape=jax.ShapeDtypeStruct((N_PAD, N_HIDDEN), _f32),
    )(x_p, deg0, deg1, W)


def _call_mm_mid(a0, a1, hs, deg0, deg1, b, W, d_out):
    return pl.pallas_call(
        _mm_mid,
        grid=(GRID,),
        in_specs=[_row_spec(N_HIDDEN), _row_spec(N_HIDDEN),
                  _row_spec(N_HIDDEN), _row_spec(16), _row_spec(16),
                  _whole_spec(1, N_HIDDEN), _whole_spec(N_HIDDEN, d_out)],
        out_specs=_row_spec(d_out),
        out_shape=jax.ShapeDtypeStruct((N_PAD, d_out), _f32),
    )(a0, a1, hs, deg0, deg1, b, W)


def _call_mm_final(a0, a1, hs, deg0, deg1, b):
    return pl.pallas_call(
        _mm_final,
        grid=(GRID,),
        in_specs=[_row_spec(N_HIDDEN), _row_spec(N_HIDDEN),
                  _row_spec(N_HIDDEN), _row_spec(16), _row_spec(16),
                  _whole_spec(1, N_HIDDEN)],
        out_specs=_row_spec(N_CLASSES),
        out_shape=jax.ShapeDtypeStruct((N_NODES, N_CLASSES), _f32),
    )(a0, a1, hs, deg0, deg1, b)


@jax.jit
def kernel(x, edge_index, W1, b1, W2, b2, W3, b3):
    src = edge_index[0].astype(jnp.int32)
    dst = edge_index[1].astype(jnp.int32)
    pad = jnp.full((E_PAD - N_EDGES,), N_NODES, jnp.int32)
    src_p = jnp.concatenate([src, pad]).reshape(NW * CH, K)
    dst_p = jnp.concatenate([dst, pad]).reshape(NW, CH, K)

    x_p = jnp.pad(x, ((0, N_PAD - N_NODES), (0, 0)))
    W3p = jnp.pad(W3, ((0, 0), (0, N_HIDDEN - N_CLASSES)))
    b1r = b1.reshape(1, N_HIDDEN)
    b2r = b2.reshape(1, N_HIDDEN)
    b3r = jnp.pad(b3, (0, N_HIDDEN - N_CLASSES)).reshape(1, N_HIDDEN)

    deg0, deg1 = _make_deg_kernel()(dst_p)

    prop128 = _make_prop_kernel(N_HIDDEN)

    hs1 = _call_mm_first(x_p, deg0, deg1, W1)
    a10, a11 = prop128(src_p, dst_p, hs1)
    hs2 = _call_mm_mid(a10, a11, hs1, deg0, deg1, b1r, W2, N_HIDDEN)
    a20, a21 = prop128(src_p, dst_p, hs2)
    hs3 = _call_mm_mid(a20, a21, hs2, deg0, deg1, b2r, W3p, N_HIDDEN)
    a30, a31 = prop128(src_p, dst_p, hs3)
    return _call_mm_final(a30, a31, hs3, deg0, deg1, b3r)


# speedup vs baseline: 26.2845x; 2.1815x over previous
import functools

import jax
import jax.numpy as jnp
from jax import lax
from jax.experimental import pallas as pl
from jax.experimental.pallas import tpu as pltpu
from jax.experimental.pallas import tpu_sc as plsc

N_NODES = 10000
N_EDGES = 320000
IN_FEATS = 128
N_HIDDEN = 128
N_CLASSES = 40

NC = 2
NS = 16
NW = NC * NS

K = 128
CH = -(-N_EDGES // (NW * K))
E_PAD = NW * CH * K
N_PAD = 10240
RPT = N_PAD // NS
BR = 512
GRID = N_PAD // BR

_f32 = jnp.float32


def _zero_fill(ref, n_rows):
    z = jnp.zeros((16,), _f32)

    def body(i, _):
        ref[i] = z
        return 0

    lax.fori_loop(0, n_rows, body, 0)


def _fill_ones(ref, n_rows):
    o = jnp.ones((16,), _f32)

    def body(i, _):
        ref[i] = o
        return 0

    lax.fori_loop(0, n_rows, body, 0)


def _make_deg_kernel():
    mesh = plsc.VectorSubcoreMesh(core_axis_name="c", subcore_axis_name="s")

    @functools.partial(
        pl.kernel,
        out_type=(
            jax.ShapeDtypeStruct((N_PAD, 16), _f32),
            jax.ShapeDtypeStruct((N_PAD, 16), _f32),
        ),
        mesh=mesh,
        scratch_types=[
            pltpu.VMEM((CH, K), jnp.int32),
            pltpu.VMEM((K, 16), _f32),
            pltpu.VMEM((16, 16), _f32),
            pltpu.VMEM_SHARED((N_PAD, 16), _f32),
        ],
    )
    def deg_kernel(dst_hbm, out0, out1, idx_v, ones_v, zeros_v, hist_sh):
        c = lax.axis_index("c")
        s = lax.axis_index("s")
        w = c * NS + s

        _fill_ones(ones_v, K)
        _zero_fill(zeros_v, 16)

        def zbody(i, _):
            pltpu.sync_copy(zeros_v, hist_sh.at[pl.ds(s * RPT + i * 16, 16)])
            return 0

        lax.fori_loop(0, RPT // 16, zbody, 0)
        plsc.subcore_barrier()

        pltpu.sync_copy(dst_hbm.at[w], idx_v)

        def body(ch, _):
            pltpu.sync_copy(ones_v, hist_sh.at[idx_v.at[ch]], add=True)
            return 0

        lax.fori_loop(0, CH, body, 0)
        plsc.subcore_barrier()

        @pl.when(c == 0)
        def _():
            pltpu.sync_copy(hist_sh.at[pl.ds(s * RPT, RPT)],
                            out0.at[pl.ds(s * RPT, RPT)])

        @pl.when(c == 1)
        def _():
            pltpu.sync_copy(hist_sh.at[pl.ds(s * RPT, RPT)],
                            out1.at[pl.ds(s * RPT, RPT)])

    return deg_kernel


def _make_prop_kernel(d_feat):
    mesh = plsc.VectorSubcoreMesh(core_axis_name="c", subcore_axis_name="s")

    @functools.partial(
        pl.kernel,
        out_type=(
            jax.ShapeDtypeStruct((N_PAD, d_feat), _f32),
            jax.ShapeDtypeStruct((N_PAD, d_feat), _f32),
        ),
        mesh=mesh,
        scratch_types=[
            pltpu.VMEM((CH, K), jnp.int32),
            pltpu.VMEM((4, K), jnp.int32),
            pltpu.VMEM((2, K, d_feat), _f32),
            pltpu.VMEM_SHARED((N_PAD, d_feat), _f32),
            pltpu.SemaphoreType.DMA,
            pltpu.SemaphoreType.DMA,
            pltpu.SemaphoreType.DMA,
            pltpu.SemaphoreType.DMA,
            pltpu.SemaphoreType.DMA,
            pltpu.SemaphoreType.DMA,
        ],
    )
    def prop_kernel(src_hbm, dst_hbm, hs_hbm, out0, out1,
                    dst_v, ibuf, msg_v, acc_sh,
                    gsem0, gsem1, isem0, isem1, isem2, isem3):
        c = lax.axis_index("c")
        s = lax.axis_index("s")
        w = c * NS + s

        pltpu.sync_copy(dst_hbm.at[w], dst_v)

        ncol = d_feat // 16
        z16 = jnp.zeros((16,), _f32)

        def zfill(i, _):
            msg_v[0, i // ncol, pl.ds((i % ncol) * 16, 16)] = z16
            return 0

        lax.fori_loop(0, K * ncol, zfill, 0)

        def zbody(i, _):
            pltpu.sync_copy(msg_v.at[0], acc_sh.at[pl.ds(s * RPT + i * K, K)])
            return 0

        lax.fori_loop(0, RPT // K, zbody, 0)

        gsems = (gsem0, gsem1)
        isems = (isem0, isem1, isem2, isem3)

        def i_start(j, q):
            pltpu.async_copy(src_hbm.at[w * CH + j], ibuf.at[q], isems[q])

        def i_wait(j, q):
            pltpu.make_async_copy(src_hbm.at[w * CH + j], ibuf.at[q],
                                  isems[q]).wait()

        def g_start(j, q, b):
            pltpu.async_copy(hs_hbm.at[ibuf.at[q]], msg_v.at[b], gsems[b])

        def g_wait(j, q, b):
            pltpu.make_async_copy(hs_hbm.at[ibuf.at[q]], msg_v.at[b],
                                  gsems[b]).wait()

        def scat(j, b):
            pltpu.sync_copy(msg_v.at[b], acc_sh.at[dst_v.at[j]], add=True)

        for q in range(4):
            i_start(q, q)
        i_wait(0, 0)
        plsc.subcore_barrier()
        g_start(0, 0, 0)

        def body(it, _):
            j0 = it * 4
            for t in range(4):
                j = j0 + t
                sl_n = (t + 1) % 4
                mb = t % 2
                mb_n = (t + 1) % 2

                @pl.when(j + 1 < CH)
                def _(j=j, sl_n=sl_n, mb_n=mb_n):
                    i_wait(j + 1, sl_n)
                    g_start(j + 1, sl_n, mb_n)

                @pl.when(j < CH)
                def _(j=j, t=t, mb=mb):
                    g_wait(j, t, mb)
                    scat(j, mb)

                @pl.when(j + 4 < CH)
                def _(j=j, t=t):
                    i_start(j + 4, t)

            return 0

        lax.fori_loop(0, -(-CH // 4), body, 0)
        plsc.subcore_barrier()

        @pl.when(c == 0)
        def _():
            pltpu.sync_copy(acc_sh.at[pl.ds(s * RPT, RPT)],
                            out0.at[pl.ds(s * RPT, RPT)])

        @pl.when(c == 1)
        def _():
            pltpu.sync_copy(acc_sh.at[pl.ds(s * RPT, RPT)],
                            out1.at[pl.ds(s * RPT, RPT)])

    return prop_kernel


def _dis(d0, d1):
    return lax.rsqrt(d0[:, 0:1] + d1[:, 0:1] + 1.0)


def _mm_first(x_ref, d0_ref, d1_ref, w_ref, o_ref):
    dis = _dis(d0_ref[...], d1_ref[...])
    h = jnp.dot(x_ref[...], w_ref[...], preferred_element_type=_f32)
    o_ref[...] = h * dis


def _mm_mid(a0_ref, a1_ref, hs_ref, d0_ref, d1_ref, b_ref, w_ref, o_ref):
    dis = _dis(d0_ref[...], d1_ref[...])
    a = (a0_ref[...] + a1_ref[...] + hs_ref[...]) * dis + b_ref[...]
    a = jnp.maximum(a, 0.0)
    h = jnp.dot(a, w_ref[...], preferred_element_type=_f32)
    o_ref[...] = h * dis


def _mm_final(a0_ref, a1_ref, hs_ref, d0_ref, d1_ref, b_ref, o_ref):
    dis = _dis(d0_ref[...], d1_ref[...])
    res = (a0_ref[...] + a1_ref[...] + hs_ref[...]) * dis + b_ref[...]
    o_ref[...] = res[:, :N_CLASSES]


def _row_spec(d):
    return pl.BlockSpec((BR, d), lambda i: (i, 0))


def _whole_spec(r, d):
    return pl.BlockSpec((r, d), lambda i: (0, 0))


def _call_mm_first(x_p, deg0, deg1, W):
    return pl.pallas_call(
        _mm_first,
        grid=(GRID,),
        in_specs=[_row_spec(IN_FEATS), _row_spec(16), _row_spec(16),
                  _whole_spec(IN_FEATS, N_HIDDEN)],
        out_specs=_row_spec(N_HIDDEN),
        out_shape=jax.ShapeDtypeStruct((N_PAD, N_HIDDEN), _f32),
    )(x_p, deg0, deg1, W)


def _call_mm_mid(a0, a1, hs, deg0, deg1, b, W, d_out):
    return pl.pallas_call(
        _mm_mid,
        grid=(GRID,),
        in_specs=[_row_spec(N_HIDDEN), _row_spec(N_HIDDEN),
                  _row_spec(N_HIDDEN), _row_spec(16), _row_spec(16),
                  _whole_spec(1, N_HIDDEN), _whole_spec(N_HIDDEN, d_out)],
        out_specs=_row_spec(d_out),
        out_shape=jax.ShapeDtypeStruct((N_PAD, d_out), _f32),
    )(a0, a1, hs, deg0, deg1, b, W)


def _call_mm_final(a0, a1, hs, deg0, deg1, b):
    return pl.pallas_call(
        _mm_final,
        grid=(GRID,),
        in_specs=[_row_spec(N_HIDDEN), _row_spec(N_HIDDEN),
                  _row_spec(N_HIDDEN), _row_spec(16), _row_spec(16),
                  _whole_spec(1, N_HIDDEN)],
        out_specs=_row_spec(N_CLASSES),
        out_shape=jax.ShapeDtypeStruct((N_NODES, N_CLASSES), _f32),
    )(a0, a1, hs, deg0, deg1, b)


@jax.jit
def kernel(x, edge_index, W1, b1, W2, b2, W3, b3):
    src = edge_index[0].astype(jnp.int32)
    dst = edge_index[1].astype(jnp.int32)
    pad = N_NODES + (jnp.arange(E_PAD - N_EDGES, dtype=jnp.int32)
                     % (N_PAD - N_NODES))
    src_p = jnp.concatenate([src, pad]).reshape(NW * CH, K)
    dst_p = jnp.concatenate([dst, pad]).reshape(NW, CH, K)

    x_p = jnp.pad(x, ((0, N_PAD - N_NODES), (0, 0)))
    W3p = jnp.pad(W3, ((0, 0), (0, N_HIDDEN - N_CLASSES)))
    b1r = b1.reshape(1, N_HIDDEN)
    b2r = b2.reshape(1, N_HIDDEN)
    b3r = jnp.pad(b3, (0, N_HIDDEN - N_CLASSES)).reshape(1, N_HIDDEN)

    deg0, deg1 = _make_deg_kernel()(dst_p)

    prop128 = _make_prop_kernel(N_HIDDEN)

    hs1 = _call_mm_first(x_p, deg0, deg1, W1)
    a10, a11 = prop128(src_p, dst_p, hs1)
    hs2 = _call_mm_mid(a10, a11, hs1, deg0, deg1, b1r, W2, N_HIDDEN)
    a20, a21 = prop128(src_p, dst_p, hs2)
    hs3 = _call_mm_mid(a20, a21, hs2, deg0, deg1, b2r, W3p, N_HIDDEN)
    a30, a31 = prop128(src_p, dst_p, hs3)
    return _call_mm_final(a30, a31, hs3, deg0, deg1, b3r)
